# baseline (speedup 1.0000x reference)
import jax
import jax.numpy as jnp
from jax.experimental import pallas as pl

KTOP = 100
NPAD = 1 << 20
CBLK = 2048
NBLK = NPAD // CBLK
BATCH = 1024
DIM = 16


def _make_mm_body(n_real):
    def _mm_body(x_ref, c_ref, o_ref):
        i = pl.program_id(0)
        x = x_ref[...]
        c = c_ref[...]
        s = jax.lax.dot_general(x, c, (((1,), (1,)), ((), ())),
                                preferred_element_type=jnp.float32)
        col = jax.lax.broadcasted_iota(jnp.int32, (BATCH, CBLK), 1) + i * CBLK
        s = jnp.where(col < n_real, s, -jnp.inf)
        o_ref[...] = s
    return _mm_body


def kernel(inputs, candidates, ids):
    n = candidates.shape[0]
    cpad = jnp.concatenate(
        [candidates, jnp.zeros((NPAD - n, DIM), jnp.float32)], axis=0)
    scores = pl.pallas_call(
        _make_mm_body(n),
        grid=(NBLK,),
        in_specs=[
            pl.BlockSpec((BATCH, DIM), lambda i: (0, 0)),
            pl.BlockSpec((CBLK, DIM), lambda i: (i, 0)),
        ],
        out_specs=pl.BlockSpec((BATCH, CBLK), lambda i: (0, i)),
        out_shape=jax.ShapeDtypeStruct((BATCH, NPAD), jnp.float32),
    )(inputs, cpad)
    ts, ti = jax.lax.top_k(scores, KTOP)
    return ts, ti


# speedup vs baseline: 8.1467x; 8.1467x over previous
import functools

import jax
import jax.numpy as jnp
from jax.experimental import pallas as pl
from jax.experimental.pallas import tpu as pltpu

KTOP = 100
NPAD = 1 << 20
CBLK = 2048
NBLK = NPAD // CBLK
BATCH = 1024
DIM = 16
NGRP = 16
NSLOT = 8
SLOTW = NGRP * NSLOT
TOTSLOT = NBLK * SLOTW
QBLK = 32
NEG = float("-inf")
IMAX = jnp.iinfo(jnp.int32).max


def _tile(m, reps):
    out = m
    total = 1
    while total < reps:
        out = jnp.concatenate([out, out], axis=1)
        total *= 2
    return out


def _fold_max(x, width):
    w = x.shape[1]
    while w > width:
        w //= 2
        x = jnp.maximum(x[:, :w], x[:, w:])
    return x


def _fold_min(x, width):
    w = x.shape[1]
    while w > width:
        w //= 2
        x = jnp.minimum(x[:, :w], x[:, w:])
    return x


def _slots_body(x_ref, c_ref, ov_ref, oi_ref, *, n_real):
    i = pl.program_id(0)
    x = x_ref[...]
    c = c_ref[...]
    s = jax.lax.dot_general(x, c, (((1,), (1,)), ((), ())),
                            preferred_element_type=jnp.float32)
    ids = jax.lax.broadcasted_iota(jnp.int32, (BATCH, CBLK), 1) + i * CBLK
    s = jnp.where(ids < n_real, s, NEG)
    vals = []
    idxs = []
    for _ in range(NSLOT):
        m = _fold_max(s, NGRP)
        mt = _tile(m, CBLK // NGRP)
        sel = s == mt
        idm = _fold_min(jnp.where(sel, ids, IMAX), NGRP)
        it = _tile(idm, CBLK // NGRP)
        s = jnp.where(sel & (ids == it), NEG, s)
        vals.append(m)
        idxs.append(idm)
    ov_ref[...] = jnp.concatenate(vals, axis=1)
    oi_ref[...] = jnp.concatenate(idxs, axis=1)


def _select_body(v_ref, i_ref, ov_ref, oi_ref, s_ref, bv_ref, bi_ref):
    s_ref[...] = v_ref[...]
    ids = i_ref[...]
    lane = jax.lax.broadcasted_iota(jnp.int32, (QBLK, 128), 1)

    def round_fn(r, carry):
        s = s_ref[...]
        m = jnp.max(s, axis=1, keepdims=True)
        sel = s == m
        idm = jnp.min(jnp.where(sel, ids, IMAX), axis=1, keepdims=True)
        s_ref[...] = jnp.where(sel & (ids == idm), NEG, s)
        hit = lane == r
        bv_ref[...] = jnp.where(hit, m, bv_ref[...])
        bi_ref[...] = jnp.where(hit, idm, bi_ref[...])
        return carry

    jax.lax.fori_loop(0, KTOP, round_fn, 0)
    ov_ref[...] = bv_ref[:, :KTOP]
    oi_ref[...] = bi_ref[:, :KTOP]


@functools.partial(jax.jit, static_argnames=("n_real",))
def _run(inputs, cpad, n_real):
    slot_v, slot_i = pl.pallas_call(
        functools.partial(_slots_body, n_real=n_real),
        grid=(NBLK,),
        in_specs=[
            pl.BlockSpec((BATCH, DIM), lambda i: (0, 0)),
            pl.BlockSpec((CBLK, DIM), lambda i: (i, 0)),
        ],
        out_specs=[
            pl.BlockSpec((BATCH, SLOTW), lambda i: (0, i)),
            pl.BlockSpec((BATCH, SLOTW), lambda i: (0, i)),
        ],
        out_shape=[
            jax.ShapeDtypeStruct((BATCH, TOTSLOT), jnp.float32),
            jax.ShapeDtypeStruct((BATCH, TOTSLOT), jnp.int32),
        ],
    )(inputs, cpad)

    ts, ti = pl.pallas_call(
        _select_body,
        grid=(BATCH // QBLK,),
        in_specs=[
            pl.BlockSpec((QBLK, TOTSLOT), lambda i: (i, 0)),
            pl.BlockSpec((QBLK, TOTSLOT), lambda i: (i, 0)),
        ],
        out_specs=[
            pl.BlockSpec((QBLK, KTOP), lambda i: (i, 0)),
            pl.BlockSpec((QBLK, KTOP), lambda i: (i, 0)),
        ],
        out_shape=[
            jax.ShapeDtypeStruct((BATCH, KTOP), jnp.float32),
            jax.ShapeDtypeStruct((BATCH, KTOP), jnp.int32),
        ],
        scratch_shapes=[
            pltpu.VMEM((QBLK, TOTSLOT), jnp.float32),
            pltpu.VMEM((QBLK, 128), jnp.float32),
            pltpu.VMEM((QBLK, 128), jnp.int32),
        ],
    )(slot_v, slot_i)
    return ts, ti


def kernel(inputs, candidates, ids):
    n = candidates.shape[0]
    cpad = jnp.concatenate(
        [candidates, jnp.zeros((NPAD - n, DIM), jnp.float32)], axis=0)
    return _run(inputs, cpad, n)


# speedup vs baseline: 9.3723x; 1.1504x over previous
import functools

import jax
import jax.numpy as jnp
from jax.experimental import pallas as pl
from jax.experimental.pallas import tpu as pltpu

KTOP = 100
NPAD = 1 << 20
CBLK = 4096
HBLK = CBLK // 2
NBLK = NPAD // CBLK
BATCH = 1024
DIM = 16
NGRP = 16
NSLOT = 8
SLOTW = NGRP * NSLOT
TOTSLOT = NBLK * SLOTW
QBLK = 32
NEG = float("-inf")
IMAX = jnp.iinfo(jnp.int32).max


def _tile(m, reps):
    out = m
    total = 1
    while total < reps:
        out = jnp.concatenate([out, out], axis=1)
        total *= 2
    return out


def _fold_max(x, width):
    w = x.shape[1]
    while w > width:
        w //= 2
        x = jnp.maximum(x[:, :w], x[:, w:])
    return x


def _fold_min(x, width):
    w = x.shape[1]
    while w > width:
        w //= 2
        x = jnp.minimum(x[:, :w], x[:, w:])
    return x


def _slots_body(x_ref, c_ref, ov_ref, oi_ref, *, n_real):
    i = pl.program_id(0)
    x = x_ref[...]
    c = c_ref[...]
    s1 = jax.lax.dot_general(x, c[:HBLK], (((1,), (1,)), ((), ())),
                             preferred_element_type=jnp.float32)
    s2 = jax.lax.dot_general(x, c[HBLK:], (((1,), (1,)), ((), ())),
                             preferred_element_type=jnp.float32)
    i1 = jax.lax.broadcasted_iota(jnp.int32, (BATCH, HBLK), 1) + i * CBLK
    i2 = i1 + HBLK
    s1 = jnp.where(i1 < n_real, s1, NEG)
    s2 = jnp.where(i2 < n_real, s2, NEG)
    vals = []
    idxs = []
    for _ in range(NSLOT):
        m = jnp.maximum(_fold_max(s1, NGRP), _fold_max(s2, NGRP))
        mt = _tile(m, HBLK // NGRP)
        sel1 = s1 == mt
        sel2 = s2 == mt
        idm = jnp.minimum(
            _fold_min(jnp.where(sel1, i1, IMAX), NGRP),
            _fold_min(jnp.where(sel2, i2, IMAX), NGRP))
        it = _tile(idm, HBLK // NGRP)
        s1 = jnp.where(sel1 & (i1 == it), NEG, s1)
        s2 = jnp.where(sel2 & (i2 == it), NEG, s2)
        vals.append(m)
        idxs.append(idm)
    ov_ref[...] = jnp.concatenate(vals, axis=1)
    oi_ref[...] = jnp.concatenate(idxs, axis=1)


def _select_body(v_ref, i_ref, ov_ref, oi_ref, s_ref, bv_ref, bi_ref):
    s_ref[...] = v_ref[...]
    ids = i_ref[...]
    lane = jax.lax.broadcasted_iota(jnp.int32, (QBLK, 128), 1)

    def round_fn(r, carry):
        s = s_ref[...]
        m = jnp.max(s, axis=1, keepdims=True)
        sel = s == m
        idm = jnp.min(jnp.where(sel, ids, IMAX), axis=1, keepdims=True)
        s_ref[...] = jnp.where(sel & (ids == idm), NEG, s)
        hit = lane == r
        bv_ref[...] = jnp.where(hit, m, bv_ref[...])
        bi_ref[...] = jnp.where(hit, idm, bi_ref[...])
        return carry

    jax.lax.fori_loop(0, KTOP, round_fn, 0)
    ov_ref[...] = bv_ref[:, :KTOP]
    oi_ref[...] = bi_ref[:, :KTOP]


@functools.partial(jax.jit, static_argnames=("n_real",))
def _run(inputs, cpad, n_real):
    slot_v, slot_i = pl.pallas_call(
        functools.partial(_slots_body, n_real=n_real),
        grid=(NBLK,),
        in_specs=[
            pl.BlockSpec((BATCH, DIM), lambda i: (0, 0)),
            pl.BlockSpec((CBLK, DIM), lambda i: (i, 0)),
        ],
        out_specs=[
            pl.BlockSpec((BATCH, SLOTW), lambda i: (0, i)),
            pl.BlockSpec((BATCH, SLOTW), lambda i: (0, i)),
        ],
        out_shape=[
            jax.ShapeDtypeStruct((BATCH, TOTSLOT), jnp.float32),
            jax.ShapeDtypeStruct((BATCH, TOTSLOT), jnp.int32),
        ],
    )(inputs, cpad)

    ts, ti = pl.pallas_call(
        _select_body,
        grid=(BATCH // QBLK,),
        in_specs=[
            pl.BlockSpec((QBLK, TOTSLOT), lambda i: (i, 0)),
            pl.BlockSpec((QBLK, TOTSLOT), lambda i: (i, 0)),
        ],
        out_specs=[
            pl.BlockSpec((QBLK, KTOP), lambda i: (i, 0)),
            pl.BlockSpec((QBLK, KTOP), lambda i: (i, 0)),
        ],
        out_shape=[
            jax.ShapeDtypeStruct((BATCH, KTOP), jnp.float32),
            jax.ShapeDtypeStruct((BATCH, KTOP), jnp.int32),
        ],
        scratch_shapes=[
            pltpu.VMEM((QBLK, TOTSLOT), jnp.float32),
            pltpu.VMEM((QBLK, 128), jnp.float32),
            pltpu.VMEM((QBLK, 128), jnp.int32),
        ],
    )(slot_v, slot_i)
    return ts, ti


def kernel(inputs, candidates, ids):
    n = candidates.shape[0]
    cpad = jnp.concatenate(
        [candidates, jnp.zeros((NPAD - n, DIM), jnp.float32)], axis=0)
    return _run(inputs, cpad, n)
